# baseline (speedup 1.0000x reference)
import jax
import jax.numpy as jnp
from jax import lax
from jax.experimental import pallas as pl
from jax.experimental.pallas import tpu as pltpu
from jax.experimental.pallas import tpu_sc as plsc

N_NODES = 50000
N_EDGES = 800000
EMB = 64
MAX_DEPTH = 20
EDGE_IN = 16
EDGE_DIM = 16

NC, NS = 2, 16
NW = NC * NS
CHUNK = 128
FULL_CHUNKS = N_NODES // CHUNK
TAIL = N_NODES - FULL_CHUNKS * CHUNK
BASE_CPW = FULL_CHUNKS // NW
EXTRA = FULL_CHUNKS % NW
TAIL_W = NW - 1


def _do_chunk(off, n, xflat, dflat, ttab, atab, dtab, out,
              xs, tid, aid, dep, bt, ba, bd, st, sa, sd, iota2):
    pltpu.sync_copy(xflat.at[pl.ds(off * 2, 2 * n)], xs)
    pltpu.sync_copy(dflat.at[pl.ds(off, n)], dep)
    for j in range(n // 16):
        sl = pl.ds(16 * j, 16)
        tid[sl] = plsc.load_gather(xs, [iota2 + (32 * j)])
        aid[sl] = plsc.load_gather(xs, [iota2 + (32 * j + 1)])
        dep[sl] = jnp.minimum(dep[sl], MAX_DEPTH)
    ct = pltpu.async_copy(ttab.at[tid], bt, st)
    ca = pltpu.async_copy(atab.at[aid], ba, sa)
    cd = pltpu.async_copy(dtab.at[dep], bd, sd)
    ct.wait()
    ca.wait()
    cd.wait()

    def add_body(i, carry):
        for r in range(4):
            for q in range(EMB // 16):
                s2 = (i * 4 + r, pl.ds(q * 16, 16))
                bt[s2] = bt[s2] + ba[s2] + bd[s2]
        return carry

    lax.fori_loop(0, n // 4, add_body, 0)
    pltpu.sync_copy(bt, out.at[pl.ds(off, n)])


def _nodes_body(xflat, dflat, ttab, atab, dtab, out,
                xs, tid, aid, dep, bt, ba, bd,
                xs_t, tid_t, aid_t, dep_t, bt_t, ba_t, bd_t,
                st, sa, sd):
    c = lax.axis_index("c")
    s = lax.axis_index("s")
    wid = s * NC + c
    iota2 = lax.iota(jnp.int32, 16) * 2
    nchunks = BASE_CPW + jnp.where(wid < EXTRA, 1, 0)

    def chunk_body(k, carry):
        off = (k * NW + wid) * CHUNK
        _do_chunk(off, CHUNK, xflat, dflat, ttab, atab, dtab, out,
                  xs, tid, aid, dep, bt, ba, bd, st, sa, sd, iota2)
        return carry

    lax.fori_loop(0, nchunks, chunk_body, 0)

    @pl.when(wid == TAIL_W)
    def _():
        _do_chunk(FULL_CHUNKS * CHUNK, TAIL, xflat, dflat, ttab, atab,
                  dtab, out, xs_t, tid_t, aid_t, dep_t, bt_t, ba_t, bd_t,
                  st, sa, sd, iota2)


def _nodes_sc(xflat, dflat, ttab, atab, dtab):
    mesh = plsc.VectorSubcoreMesh(core_axis_name="c", subcore_axis_name="s")
    return pl.kernel(
        _nodes_body,
        out_type=jax.ShapeDtypeStruct((N_NODES, EMB), jnp.float32),
        mesh=mesh,
        scratch_types=[
            pltpu.VMEM((2 * CHUNK,), jnp.int32),
            pltpu.VMEM((CHUNK,), jnp.int32),
            pltpu.VMEM((CHUNK,), jnp.int32),
            pltpu.VMEM((CHUNK,), jnp.int32),
            pltpu.VMEM((CHUNK, EMB), jnp.float32),
            pltpu.VMEM((CHUNK, EMB), jnp.float32),
            pltpu.VMEM((CHUNK, EMB), jnp.float32),
            pltpu.VMEM((2 * TAIL,), jnp.int32),
            pltpu.VMEM((TAIL,), jnp.int32),
            pltpu.VMEM((TAIL,), jnp.int32),
            pltpu.VMEM((TAIL,), jnp.int32),
            pltpu.VMEM((TAIL, EMB), jnp.float32),
            pltpu.VMEM((TAIL, EMB), jnp.float32),
            pltpu.VMEM((TAIL, EMB), jnp.float32),
            pltpu.SemaphoreType.DMA,
            pltpu.SemaphoreType.DMA,
            pltpu.SemaphoreType.DMA,
        ],
        compiler_params=pltpu.CompilerParams(
            use_tc_tiling_on_sc=False, needs_layout_passes=False),
    )(xflat, dflat, ttab, atab, dtab)


EDGE_ROWS = N_EDGES // 8
EDGE_BLK = 2000


def _edge_body(w_ref, x_ref, o_ref):
    w = w_ref[...]
    wt = jnp.tile(w, (8, 8))
    ri = lax.broadcasted_iota(jnp.int32, (128, 128), 0) // EDGE_IN
    ci = lax.broadcasted_iota(jnp.int32, (128, 128), 1) // EDGE_DIM
    wb = jnp.where(ri == ci, wt, 0.0)
    o_ref[...] = jnp.dot(x_ref[...], wb, preferred_element_type=jnp.float32)


def _edges_tc(edges2, W_edge):
    return pl.pallas_call(
        _edge_body,
        grid=(EDGE_ROWS // EDGE_BLK,),
        in_specs=[
            pl.BlockSpec((EDGE_IN, EDGE_DIM), lambda i: (0, 0)),
            pl.BlockSpec((EDGE_BLK, 128), lambda i: (i, 0)),
        ],
        out_specs=pl.BlockSpec((EDGE_BLK, 128), lambda i: (i, 0)),
        out_shape=jax.ShapeDtypeStruct((EDGE_ROWS, 128), jnp.float32),
    )(W_edge, edges2)


def kernel(x, depth, edges, type_encoder, attribute_encoder, depth_encoder,
           W_edge):
    xflat = x.reshape(N_NODES * 2)
    dflat = depth.reshape(N_NODES)
    nodes = _nodes_sc(xflat, dflat, type_encoder, attribute_encoder,
                      depth_encoder)
    edges2 = edges.reshape(EDGE_ROWS, 128)
    edges_out = _edges_tc(edges2, W_edge).reshape(N_EDGES, EDGE_DIM)
    return (nodes, edges_out)


# speedup vs baseline: 4.2976x; 4.2976x over previous
import jax
import jax.numpy as jnp
from jax import lax
from jax.experimental import pallas as pl
from jax.experimental.pallas import tpu as pltpu
from jax.experimental.pallas import tpu_sc as plsc

N_NODES = 50000
N_EDGES = 800000
EMB = 64
NUM_TYPES = 1000
NUM_ATTRS = 100000
MAX_DEPTH = 20
EDGE_IN = 16
EDGE_DIM = 16

NC, NS = 2, 16
NW = NC * NS
COLS_PER_W = EMB // NW
SEG = 2048
FULL_SEGS = N_NODES // SEG
TAIL = N_NODES - FULL_SEGS * SEG
TAIL_GROUPS = TAIL // 16


def _do_segment(off, n, c, xT, dT, outT, tseg, aseg, dseg, oseg,
                acol, tcol, dtab):
    pltpu.sync_copy(xT.at[0, pl.ds(off, n)], tseg.at[pl.ds(0, n)])
    pltpu.sync_copy(xT.at[1, pl.ds(off, n)], aseg.at[pl.ds(0, n)])
    pltpu.sync_copy(dT.at[0, pl.ds(off, n)], dseg.at[pl.ds(0, n)])
    cvec = jnp.full((16,), c, dtype=jnp.int32)

    def group(i, carry):
        for u in range(4):
            sl = pl.ds(i * 64 + u * 16, 16)
            d16 = jnp.minimum(dseg[sl], MAX_DEPTH)
            v = (plsc.load_gather(acol, [aseg[sl]])
                 + plsc.load_gather(tcol, [tseg[sl]])
                 + plsc.load_gather(dtab, [cvec, d16]))
            oseg[sl] = v
        return carry

    lax.fori_loop(0, n // 64, group, 0)
    rem = (n % 64) // 16
    for u in range(rem):
        sl = pl.ds((n // 64) * 64 + u * 16, 16)
        d16 = jnp.minimum(dseg[sl], MAX_DEPTH)
        v = (plsc.load_gather(acol, [aseg[sl]])
             + plsc.load_gather(tcol, [tseg[sl]])
             + plsc.load_gather(dtab, [cvec, d16]))
        oseg[sl] = v
    pltpu.sync_copy(oseg.at[pl.ds(0, n)], outT.at[c, pl.ds(off, n)])


def _nodes_body(xT, dT, tT, aT, dthT, outT,
                acol, tcol, dtab, tseg, aseg, dseg, oseg):
    cc = lax.axis_index("c")
    ss = lax.axis_index("s")
    wid = ss * NC + cc
    pltpu.sync_copy(dthT, dtab)

    for q in range(COLS_PER_W):
        c = wid * COLS_PER_W + q
        pltpu.sync_copy(aT.at[c], acol)
        pltpu.sync_copy(tT.at[c], tcol)

        def seg_body(s, carry):
            _do_segment(s * SEG, SEG, c, xT, dT, outT,
                        tseg, aseg, dseg, oseg, acol, tcol, dtab)
            return carry

        lax.fori_loop(0, FULL_SEGS, seg_body, 0)
        _do_segment(FULL_SEGS * SEG, TAIL, c, xT, dT, outT,
                    tseg, aseg, dseg, oseg, acol, tcol, dtab)


def _nodes_sc(xT, dT, tT, aT, dthT):
    mesh = plsc.VectorSubcoreMesh(core_axis_name="c", subcore_axis_name="s")
    return pl.kernel(
        _nodes_body,
        out_type=jax.ShapeDtypeStruct((EMB, N_NODES), jnp.float32),
        mesh=mesh,
        scratch_types=[
            pltpu.VMEM((NUM_ATTRS,), jnp.float32),
            pltpu.VMEM((NUM_TYPES,), jnp.float32),
            pltpu.VMEM((EMB, MAX_DEPTH + 1), jnp.float32),
            pltpu.VMEM((SEG,), jnp.int32),
            pltpu.VMEM((SEG,), jnp.int32),
            pltpu.VMEM((SEG,), jnp.int32),
            pltpu.VMEM((SEG,), jnp.float32),
        ],
        compiler_params=pltpu.CompilerParams(
            use_tc_tiling_on_sc=False, needs_layout_passes=False),
    )(xT, dT, tT, aT, dthT)


EDGE_BLK = 16000


def _edge_body(w_ref, x_ref, o_ref):
    o_ref[...] = lax.dot_general(
        w_ref[...], x_ref[...], (((0,), (0,)), ((), ())),
        preferred_element_type=jnp.float32)


def _edges_tc(eT, W_edge):
    return pl.pallas_call(
        _edge_body,
        grid=(N_EDGES // EDGE_BLK,),
        in_specs=[
            pl.BlockSpec((EDGE_IN, EDGE_DIM), lambda i: (0, 0)),
            pl.BlockSpec((EDGE_IN, EDGE_BLK), lambda i: (0, i)),
        ],
        out_specs=pl.BlockSpec((EDGE_DIM, EDGE_BLK), lambda i: (0, i)),
        out_shape=jax.ShapeDtypeStruct((EDGE_DIM, N_EDGES), jnp.float32),
    )(W_edge, eT)


def kernel(x, depth, edges, type_encoder, attribute_encoder, depth_encoder,
           W_edge):
    nodesT = _nodes_sc(x.T, depth.T, type_encoder.T, attribute_encoder.T,
                       depth_encoder.T)
    edges_outT = _edges_tc(edges.T, W_edge)
    return (nodesT.T, edges_outT.T)


# speedup vs baseline: 6.8288x; 1.5890x over previous
import jax
import jax.numpy as jnp
from jax import lax
from jax.experimental import pallas as pl
from jax.experimental.pallas import tpu as pltpu
from jax.experimental.pallas import tpu_sc as plsc

N_NODES = 50000
N_EDGES = 800000
EMB = 64
NUM_TYPES = 1000
NUM_ATTRS = 100000
MAX_DEPTH = 20
EDGE_IN = 16
EDGE_DIM = 16

NC, NS = 2, 16
NW = NC * NS
COLS_PER_W = EMB // NW
SEG = 3200
FULL_SEGS = N_NODES // SEG
TAIL = N_NODES - FULL_SEGS * SEG
NSEGS = FULL_SEGS + 1


def _gather_groups(n, cvec, tseg, aseg, dseg, oseg, acol, tcol, dtab):
    def group(i, carry):
        for u in range(4):
            sl = pl.ds(i * 64 + u * 16, 16)
            d16 = jnp.minimum(dseg[sl], MAX_DEPTH)
            v = (plsc.load_gather(acol, [aseg[sl]])
                 + plsc.load_gather(tcol, [tseg[sl]])
                 + plsc.load_gather(dtab, [cvec, d16]))
            oseg[sl] = v
        return carry

    lax.fori_loop(0, n // 64, group, 0)
    base = (n // 64) * 64
    for u in range((n % 64) // 16):
        sl = pl.ds(base + u * 16, 16)
        d16 = jnp.minimum(dseg[sl], MAX_DEPTH)
        v = (plsc.load_gather(acol, [aseg[sl]])
             + plsc.load_gather(tcol, [tseg[sl]])
             + plsc.load_gather(dtab, [cvec, d16]))
        oseg[sl] = v


def _seg_len(s):
    return SEG if s < FULL_SEGS else TAIL


def _nodes_body(xT, dT, tT, aT, dthT, outT,
                acol, tcol, dtab,
                ts0, ts1, as0, as1, ds0, ds1, os0, os1,
                si0, si1, sw0, sw1):
    cc = lax.axis_index("c")
    ss = lax.axis_index("s")
    wid = ss * NC + cc
    pltpu.sync_copy(dthT, dtab)

    tbufs = (ts0, ts1)
    abufs = (as0, as1)
    dbufs = (ds0, ds1)
    obufs = (os0, os1)
    isems = (si0, si1)
    osems = (sw0, sw1)

    def issue_idx(s):
        b = s & 1
        n = _seg_len(s)
        off = s * SEG
        return [
            pltpu.async_copy(xT.at[0, pl.ds(off, n)],
                             tbufs[b].at[pl.ds(0, n)], isems[b]),
            pltpu.async_copy(xT.at[1, pl.ds(off, n)],
                             abufs[b].at[pl.ds(0, n)], isems[b]),
            pltpu.async_copy(dT.at[0, pl.ds(off, n)],
                             dbufs[b].at[pl.ds(0, n)], isems[b]),
        ]

    for q in range(COLS_PER_W):
        c = wid * COLS_PER_W + q
        pltpu.sync_copy(aT.at[c], acol)
        pltpu.sync_copy(tT.at[c], tcol)
        cvec = jnp.full((16,), c, dtype=jnp.int32)
        out_pending = [None, None]
        idx_pending = issue_idx(0)
        for s in range(NSEGS):
            b = s & 1
            n = _seg_len(s)
            for h in idx_pending:
                h.wait()
            if s + 1 < NSEGS:
                idx_pending = issue_idx(s + 1)
            if out_pending[b] is not None:
                out_pending[b].wait()
            _gather_groups(n, cvec, tbufs[b], abufs[b], dbufs[b], obufs[b],
                           acol, tcol, dtab)
            out_pending[b] = pltpu.async_copy(
                obufs[b].at[pl.ds(0, n)], outT.at[c, pl.ds(s * SEG, n)],
                osems[b])
        for b in (0, 1):
            if out_pending[b] is not None:
                out_pending[b].wait()


def _nodes_sc(xT, dT, tT, aT, dthT):
    mesh = plsc.VectorSubcoreMesh(core_axis_name="c", subcore_axis_name="s")
    return pl.kernel(
        _nodes_body,
        out_type=jax.ShapeDtypeStruct((EMB, N_NODES), jnp.float32),
        mesh=mesh,
        scratch_types=[
            pltpu.VMEM((NUM_ATTRS,), jnp.float32),
            pltpu.VMEM((NUM_TYPES,), jnp.float32),
            pltpu.VMEM((EMB, MAX_DEPTH + 1), jnp.float32),
            pltpu.VMEM((SEG,), jnp.int32),
            pltpu.VMEM((SEG,), jnp.int32),
            pltpu.VMEM((SEG,), jnp.int32),
            pltpu.VMEM((SEG,), jnp.int32),
            pltpu.VMEM((SEG,), jnp.int32),
            pltpu.VMEM((SEG,), jnp.int32),
            pltpu.VMEM((SEG,), jnp.float32),
            pltpu.VMEM((SEG,), jnp.float32),
            pltpu.SemaphoreType.DMA,
            pltpu.SemaphoreType.DMA,
            pltpu.SemaphoreType.DMA,
            pltpu.SemaphoreType.DMA,
        ],
        compiler_params=pltpu.CompilerParams(
            use_tc_tiling_on_sc=False, needs_layout_passes=False),
    )(xT, dT, tT, aT, dthT)


EDGE_BLK = 16000


def _edge_body(w_ref, x_ref, o_ref):
    o_ref[...] = lax.dot_general(
        w_ref[...], x_ref[...], (((0,), (0,)), ((), ())),
        preferred_element_type=jnp.float32)


def _edges_tc(eT, W_edge):
    return pl.pallas_call(
        _edge_body,
        grid=(N_EDGES // EDGE_BLK,),
        in_specs=[
            pl.BlockSpec((EDGE_IN, EDGE_DIM), lambda i: (0, 0)),
            pl.BlockSpec((EDGE_IN, EDGE_BLK), lambda i: (0, i)),
        ],
        out_specs=pl.BlockSpec((EDGE_DIM, EDGE_BLK), lambda i: (0, i)),
        out_shape=jax.ShapeDtypeStruct((EDGE_DIM, N_EDGES), jnp.float32),
    )(W_edge, eT)


def kernel(x, depth, edges, type_encoder, attribute_encoder, depth_encoder,
           W_edge):
    nodesT = _nodes_sc(x.T, depth.T, type_encoder.T, attribute_encoder.T,
                       depth_encoder.T)
    edges_outT = _edges_tc(edges.T, W_edge)
    return (nodesT.T, edges_outT.T)


# speedup vs baseline: 6.9864x; 1.0231x over previous
import jax
import jax.numpy as jnp
from jax import lax
from jax.experimental import pallas as pl
from jax.experimental.pallas import tpu as pltpu
from jax.experimental.pallas import tpu_sc as plsc

N_NODES = 50000
N_EDGES = 800000
EMB = 64
NUM_TYPES = 1000
NUM_ATTRS = 100000
MAX_DEPTH = 20
EDGE_IN = 16
EDGE_DIM = 16

NC, NS = 2, 16
NW = NC * NS
COLS_PER_W = EMB // NW
SEG = 3200
FULL_SEGS = N_NODES // SEG
TAIL = N_NODES - FULL_SEGS * SEG
NSEGS = FULL_SEGS + 1


def _gather_groups(n, cvec, tseg, aseg, dseg, oseg, acol, tcol, dtab):
    @plsc.parallel_loop(0, n // 64, 1, unroll=1)
    def group(i):
        for u in range(4):
            sl = pl.ds(i * 64 + u * 16, 16)
            d16 = jnp.minimum(dseg[sl], MAX_DEPTH)
            v = (plsc.load_gather(acol, [aseg[sl]])
                 + plsc.load_gather(tcol, [tseg[sl]])
                 + plsc.load_gather(dtab, [cvec, d16]))
            oseg[sl] = v
    base = (n // 64) * 64
    for u in range((n % 64) // 16):
        sl = pl.ds(base + u * 16, 16)
        d16 = jnp.minimum(dseg[sl], MAX_DEPTH)
        v = (plsc.load_gather(acol, [aseg[sl]])
             + plsc.load_gather(tcol, [tseg[sl]])
             + plsc.load_gather(dtab, [cvec, d16]))
        oseg[sl] = v


def _seg_len(s):
    return SEG if s < FULL_SEGS else TAIL


def _nodes_body(xT, dT, tT, aT, dthT, outT,
                acol, tcol, dtab,
                ts0, ts1, as0, as1, ds0, ds1, os0, os1,
                si0, si1, sw0, sw1):
    cc = lax.axis_index("c")
    ss = lax.axis_index("s")
    wid = ss * NC + cc
    pltpu.sync_copy(dthT, dtab)

    tbufs = (ts0, ts1)
    abufs = (as0, as1)
    dbufs = (ds0, ds1)
    obufs = (os0, os1)
    isems = (si0, si1)
    osems = (sw0, sw1)

    def issue_idx(s):
        b = s & 1
        n = _seg_len(s)
        off = s * SEG
        return [
            pltpu.async_copy(xT.at[0, pl.ds(off, n)],
                             tbufs[b].at[pl.ds(0, n)], isems[b]),
            pltpu.async_copy(xT.at[1, pl.ds(off, n)],
                             abufs[b].at[pl.ds(0, n)], isems[b]),
            pltpu.async_copy(dT.at[0, pl.ds(off, n)],
                             dbufs[b].at[pl.ds(0, n)], isems[b]),
        ]

    for q in range(COLS_PER_W):
        c = wid * COLS_PER_W + q
        pltpu.sync_copy(aT.at[c], acol)
        pltpu.sync_copy(tT.at[c], tcol)
        cvec = jnp.full((16,), c, dtype=jnp.int32)
        out_pending = [None, None]
        idx_pending = issue_idx(0)
        for s in range(NSEGS):
            b = s & 1
            n = _seg_len(s)
            for h in idx_pending:
                h.wait()
            if s + 1 < NSEGS:
                idx_pending = issue_idx(s + 1)
            if out_pending[b] is not None:
                out_pending[b].wait()
            _gather_groups(n, cvec, tbufs[b], abufs[b], dbufs[b], obufs[b],
                           acol, tcol, dtab)
            out_pending[b] = pltpu.async_copy(
                obufs[b].at[pl.ds(0, n)], outT.at[c, pl.ds(s * SEG, n)],
                osems[b])
        for b in (0, 1):
            if out_pending[b] is not None:
                out_pending[b].wait()


def _nodes_sc(xT, dT, tT, aT, dthT):
    mesh = plsc.VectorSubcoreMesh(core_axis_name="c", subcore_axis_name="s")
    return pl.kernel(
        _nodes_body,
        out_type=jax.ShapeDtypeStruct((EMB, N_NODES), jnp.float32),
        mesh=mesh,
        scratch_types=[
            pltpu.VMEM((NUM_ATTRS,), jnp.float32),
            pltpu.VMEM((NUM_TYPES,), jnp.float32),
            pltpu.VMEM((EMB, MAX_DEPTH + 1), jnp.float32),
            pltpu.VMEM((SEG,), jnp.int32),
            pltpu.VMEM((SEG,), jnp.int32),
            pltpu.VMEM((SEG,), jnp.int32),
            pltpu.VMEM((SEG,), jnp.int32),
            pltpu.VMEM((SEG,), jnp.int32),
            pltpu.VMEM((SEG,), jnp.int32),
            pltpu.VMEM((SEG,), jnp.float32),
            pltpu.VMEM((SEG,), jnp.float32),
            pltpu.SemaphoreType.DMA,
            pltpu.SemaphoreType.DMA,
            pltpu.SemaphoreType.DMA,
            pltpu.SemaphoreType.DMA,
        ],
        compiler_params=pltpu.CompilerParams(
            use_tc_tiling_on_sc=False, needs_layout_passes=False),
    )(xT, dT, tT, aT, dthT)


EDGE_BLK = 16000


def _edge_body(w_ref, x_ref, o_ref):
    o_ref[...] = lax.dot_general(
        w_ref[...], x_ref[...], (((0,), (0,)), ((), ())),
        preferred_element_type=jnp.float32)


def _edges_tc(eT, W_edge):
    return pl.pallas_call(
        _edge_body,
        grid=(N_EDGES // EDGE_BLK,),
        in_specs=[
            pl.BlockSpec((EDGE_IN, EDGE_DIM), lambda i: (0, 0)),
            pl.BlockSpec((EDGE_IN, EDGE_BLK), lambda i: (0, i)),
        ],
        out_specs=pl.BlockSpec((EDGE_DIM, EDGE_BLK), lambda i: (0, i)),
        out_shape=jax.ShapeDtypeStruct((EDGE_DIM, N_EDGES), jnp.float32),
    )(W_edge, eT)


def kernel(x, depth, edges, type_encoder, attribute_encoder, depth_encoder,
           W_edge):
    nodesT = _nodes_sc(x.T, depth.T, type_encoder.T, attribute_encoder.T,
                       depth_encoder.T)
    edges_outT = _edges_tc(edges.T, W_edge)
    return (nodesT.T, edges_outT.T)


# speedup vs baseline: 6.9957x; 1.0013x over previous
import jax
import jax.numpy as jnp
from jax import lax
from jax.experimental import pallas as pl
from jax.experimental.pallas import tpu as pltpu
from jax.experimental.pallas import tpu_sc as plsc

N_NODES = 50000
N_EDGES = 800000
EMB = 64
NUM_TYPES = 1000
NUM_ATTRS = 100000
MAX_DEPTH = 20
EDGE_IN = 16
EDGE_DIM = 16

NC, NS = 2, 16
NW = NC * NS
COLS_PER_W = EMB // NW
SEG = 3200
FULL_SEGS = N_NODES // SEG
TAIL = N_NODES - FULL_SEGS * SEG
NSEGS = FULL_SEGS + 1


def _gather_groups(n, cvec, tseg, aseg, dseg, oseg, acol, tcol, dtab):
    @plsc.parallel_loop(0, n // 64, 1, unroll=2)
    def group(i):
        for u in range(4):
            sl = pl.ds(i * 64 + u * 16, 16)
            d16 = jnp.minimum(dseg[sl], MAX_DEPTH)
            v = (plsc.load_gather(acol, [aseg[sl]])
                 + plsc.load_gather(tcol, [tseg[sl]])
                 + plsc.load_gather(dtab, [cvec, d16]))
            oseg[sl] = v
    base = (n // 64) * 64
    for u in range((n % 64) // 16):
        sl = pl.ds(base + u * 16, 16)
        d16 = jnp.minimum(dseg[sl], MAX_DEPTH)
        v = (plsc.load_gather(acol, [aseg[sl]])
             + plsc.load_gather(tcol, [tseg[sl]])
             + plsc.load_gather(dtab, [cvec, d16]))
        oseg[sl] = v


def _seg_len(s):
    return SEG if s < FULL_SEGS else TAIL


def _nodes_body(xT, dT, tT, aT, dthT, outT,
                acol, tcol, dtab,
                ts0, ts1, as0, as1, ds0, ds1, os0, os1,
                si0, si1, sw0, sw1):
    cc = lax.axis_index("c")
    ss = lax.axis_index("s")
    wid = ss * NC + cc
    pltpu.sync_copy(dthT, dtab)

    tbufs = (ts0, ts1)
    abufs = (as0, as1)
    dbufs = (ds0, ds1)
    obufs = (os0, os1)
    isems = (si0, si1)
    osems = (sw0, sw1)

    def issue_idx(s):
        b = s & 1
        n = _seg_len(s)
        off = s * SEG
        return [
            pltpu.async_copy(xT.at[0, pl.ds(off, n)],
                             tbufs[b].at[pl.ds(0, n)], isems[b]),
            pltpu.async_copy(xT.at[1, pl.ds(off, n)],
                             abufs[b].at[pl.ds(0, n)], isems[b]),
            pltpu.async_copy(dT.at[0, pl.ds(off, n)],
                             dbufs[b].at[pl.ds(0, n)], isems[b]),
        ]

    def col_body(q, carry):
        c = wid * COLS_PER_W + q
        pltpu.sync_copy(aT.at[c], acol)
        pltpu.sync_copy(tT.at[c], tcol)
        cvec = jnp.full((16,), c, dtype=jnp.int32)
        out_pending = [None, None]
        idx_pending = issue_idx(0)
        for s in range(NSEGS):
            b = s & 1
            n = _seg_len(s)
            for h in idx_pending:
                h.wait()
            if s + 1 < NSEGS:
                idx_pending = issue_idx(s + 1)
            if out_pending[b] is not None:
                out_pending[b].wait()
            _gather_groups(n, cvec, tbufs[b], abufs[b], dbufs[b], obufs[b],
                           acol, tcol, dtab)
            out_pending[b] = pltpu.async_copy(
                obufs[b].at[pl.ds(0, n)], outT.at[c, pl.ds(s * SEG, n)],
                osems[b])
        for b in (0, 1):
            if out_pending[b] is not None:
                out_pending[b].wait()
        return carry

    lax.fori_loop(0, COLS_PER_W, col_body, 0)


def _nodes_sc(xT, dT, tT, aT, dthT):
    mesh = plsc.VectorSubcoreMesh(core_axis_name="c", subcore_axis_name="s")
    return pl.kernel(
        _nodes_body,
        out_type=jax.ShapeDtypeStruct((EMB, N_NODES), jnp.float32),
        mesh=mesh,
        scratch_types=[
            pltpu.VMEM((NUM_ATTRS,), jnp.float32),
            pltpu.VMEM((NUM_TYPES,), jnp.float32),
            pltpu.VMEM((EMB, MAX_DEPTH + 1), jnp.float32),
            pltpu.VMEM((SEG,), jnp.int32),
            pltpu.VMEM((SEG,), jnp.int32),
            pltpu.VMEM((SEG,), jnp.int32),
            pltpu.VMEM((SEG,), jnp.int32),
            pltpu.VMEM((SEG,), jnp.int32),
            pltpu.VMEM((SEG,), jnp.int32),
            pltpu.VMEM((SEG,), jnp.float32),
            pltpu.VMEM((SEG,), jnp.float32),
            pltpu.SemaphoreType.DMA,
            pltpu.SemaphoreType.DMA,
            pltpu.SemaphoreType.DMA,
            pltpu.SemaphoreType.DMA,
        ],
        compiler_params=pltpu.CompilerParams(
            use_tc_tiling_on_sc=False, needs_layout_passes=False),
    )(xT, dT, tT, aT, dthT)


EDGE_BLK = 16000


def _edge_body(w_ref, x_ref, o_ref):
    o_ref[...] = lax.dot_general(
        w_ref[...], x_ref[...], (((0,), (0,)), ((), ())),
        preferred_element_type=jnp.float32)


def _edges_tc(eT, W_edge):
    return pl.pallas_call(
        _edge_body,
        grid=(N_EDGES // EDGE_BLK,),
        in_specs=[
            pl.BlockSpec((EDGE_IN, EDGE_DIM), lambda i: (0, 0)),
            pl.BlockSpec((EDGE_IN, EDGE_BLK), lambda i: (0, i)),
        ],
        out_specs=pl.BlockSpec((EDGE_DIM, EDGE_BLK), lambda i: (0, i)),
        out_shape=jax.ShapeDtypeStruct((EDGE_DIM, N_EDGES), jnp.float32),
    )(W_edge, eT)


def kernel(x, depth, edges, type_encoder, attribute_encoder, depth_encoder,
           W_edge):
    nodesT = _nodes_sc(x.T, depth.T, type_encoder.T, attribute_encoder.T,
                       depth_encoder.T)
    edges_outT = _edges_tc(edges.T, W_edge)
    return (nodesT.T, edges_outT.T)
